# baseline (speedup 1.0000x reference)
import jax
import jax.numpy as jnp
from jax.experimental import pallas as pl
from jax.experimental.pallas import tpu as pltpu

TOTAL = 16384
D = 2048
SEG = 2048
NSEG = TOTAL // SEG
BM = 1024
NHALF = TOTAL // BM
NSLOT = 4

_NT = (((1,), (1,)), ((), ()))


def _fused_body(x_hbm, w_ref, b_ref, o_ref, ring_ref, xm_ref, sems):
    s = pl.program_id(0)
    m = pl.program_id(1)
    q = 2 * s + m

    def copy(h):
        return pltpu.make_async_copy(
            x_hbm.at[pl.ds(h * BM, BM), :], ring_ref.at[h % NSLOT],
            sems.at[h % NSLOT])

    @pl.when(q == 0)
    def _():
        copy(0).start()
        copy(1).start()
        copy(2).start()

    @pl.when(q + 3 < NHALF)
    def _():
        copy(q + 3).start()

    @pl.when(m == 0)
    def _():
        copy(q).wait()
        copy(q + 1).wait()
        xm_ref[...] = (
            jnp.sum(ring_ref[q % NSLOT], axis=0, keepdims=True)
            + jnp.sum(ring_ref[(q + 1) % NSLOT], axis=0, keepdims=True)
        ) * (1.0 / SEG)

    xc = (ring_ref[q % NSLOT] - xm_ref[...]).astype(jnp.bfloat16)
    o_ref[...] = jax.lax.dot_general(
        xc, w_ref[...], dimension_numbers=_NT,
        preferred_element_type=jnp.float32,
    ) + b_ref[...]


def kernel(x, W, b, l):
    b_eff = (b + (jnp.asarray(l) - SEG).astype(jnp.float32)).reshape(1, D)
    W_bf = W.astype(jnp.bfloat16)

    out = pl.pallas_call(
        _fused_body,
        grid=(NSEG, SEG // BM),
        in_specs=[
            pl.BlockSpec(memory_space=pltpu.MemorySpace.HBM),
            pl.BlockSpec((D, D), lambda s, m: (0, 0)),
            pl.BlockSpec((1, D), lambda s, m: (0, 0)),
        ],
        out_specs=pl.BlockSpec((BM, D), lambda s, m: (2 * s + m, 0)),
        out_shape=jax.ShapeDtypeStruct((TOTAL, D), jnp.float32),
        scratch_shapes=[
            pltpu.VMEM((NSLOT, BM, D), jnp.float32),
            pltpu.VMEM((1, D), jnp.float32),
            pltpu.SemaphoreType.DMA((NSLOT,)),
        ],
        compiler_params=pltpu.CompilerParams(
            vmem_limit_bytes=64 * 1024 * 1024,
        ),
    )(x, W_bf, b_eff)
    return out
